# Initial kernel scaffold; baseline (speedup 1.0000x reference)
import jax
import jax.numpy as jnp
from jax.experimental import pallas as pl


def kernel(input_ids, attention_mask, labels, word_emb, pos_emb, type_emb, emb_ln_g, emb_ln_b, Wg1, bg1, Wg2, bg2, W1, b1, W2, b2, head_w, head_b, head_ln_g, head_ln_b, dec_w, dec_b):
    raise NotImplementedError("write your pallas kernel here")


# speedup vs baseline: 1.2315x; 1.2315x over previous
import functools

import jax
import jax.numpy as jnp
from jax import lax
from jax.experimental import pallas as pl
from jax.experimental.pallas import tpu as pltpu
from jax.experimental.pallas import tpu_sc as plsc

V = 30522
D = 768
G1, G2 = 2, 4
E = 8
K = 5
S = 2048
VT = 512
NV = (V + VT - 1) // VT


def _make_sc_gather():
    info = plsc.get_sparse_core_info()
    nc, ns = info.num_cores, info.num_subcores
    nw = nc * ns
    b_per_w = S // nw
    mesh = plsc.VectorSubcoreMesh(core_axis_name="c", subcore_axis_name="s")

    @functools.partial(
        pl.kernel,
        out_type=jax.ShapeDtypeStruct((S, D), jnp.float32),
        mesh=mesh,
        scratch_types=[
            pltpu.VMEM((b_per_w,), jnp.int32),
            pltpu.VMEM((b_per_w, D), jnp.float32),
            pltpu.SemaphoreType.DMA,
        ],
    )
    def gather_k(table_hbm, idx_hbm, out_hbm, idx_v, rows_v, sem):
        wid = lax.axis_index("s") * nc + lax.axis_index("c")
        base = wid * b_per_w
        pltpu.sync_copy(idx_hbm.at[pl.ds(base, b_per_w)], idx_v)
        pltpu.async_copy(table_hbm.at[idx_v], rows_v, sem).wait()
        pltpu.sync_copy(rows_v, out_hbm.at[pl.ds(base, b_per_w)])

    return gather_k


def _make_moe_body(first):
    def body(*refs):
        if first:
            (x_ref, const_ref, lng_ref, lnb_ref, wg1_ref, bg1_ref, wg2_ref,
             bg2_ref, w1_ref, b1_ref, w2_ref, b2_ref, mask_ref, out_ref,
             xn_ref, gd_ref) = refs
        else:
            (x_ref, wg1_ref, bg1_ref, wg2_ref, bg2_ref, w1_ref, b1_ref,
             w2_ref, b2_ref, mask_ref, out_ref, gd_ref) = refs
        e = pl.program_id(0)
        col = lax.broadcasted_iota(jnp.int32, (S, E), 1)

        @pl.when(e == 0)
        def _prologue():
            if first:
                v = x_ref[...] + const_ref[...]
                mu = jnp.mean(v, axis=1, keepdims=True)
                var = jnp.mean((v - mu) ** 2, axis=1, keepdims=True)
                xn = ((v - mu) * lax.rsqrt(var + 1e-5) * lng_ref[...]
                      + lnb_ref[...])
                xn_ref[...] = xn
            else:
                xn = x_ref[...]
            l1 = jnp.dot(xn, wg1_ref[...],
                         preferred_element_type=jnp.float32) + bg1_ref[...]
            l2 = jnp.dot(xn, wg2_ref[...],
                         preferred_element_type=jnp.float32) + bg2_ref[...]
            s = jnp.concatenate([l1[:, i:i + 1] + l2 for i in range(G1)],
                                axis=1)
            rank = jnp.zeros(s.shape, jnp.float32)
            for j in range(E):
                sj = s[:, j:j + 1]
                rank += jnp.where(sj > s, 1.0, 0.0)
                rank += jnp.where((sj == s) & (j < col), 1.0, 0.0)
            sm = jnp.where(rank < K, s, -jnp.inf)
            mx = jnp.max(sm, axis=1, keepdims=True)
            p = jnp.exp(sm - mx)
            gd_ref[...] = p / jnp.sum(p, axis=1, keepdims=True)

        xn = xn_ref[...] if first else x_ref[...]
        h = jnp.dot(xn, w1_ref[0],
                    preferred_element_type=jnp.float32) + b1_ref[0]
        h = jax.nn.gelu(h)
        y = jnp.dot(h, w2_ref[0],
                    preferred_element_type=jnp.float32) + b2_ref[0]
        g = jnp.sum(jnp.where(col == e, gd_ref[...], 0.0), axis=1,
                    keepdims=True)
        gy = (g.astype(jnp.bfloat16).astype(jnp.float32)
              * y.astype(jnp.bfloat16).astype(jnp.float32))

        @pl.when(e == 0)
        def _init():
            out_ref[...] = gy

        @pl.when(e > 0)
        def _acc():
            out_ref[...] += gy

        @pl.when(e == E - 1)
        def _mask():
            out_ref[...] = out_ref[...] * mask_ref[...]

    return body


def _moe_pallas_args(first):
    full2d = pl.BlockSpec((S, D), lambda e: (0, 0))
    row = pl.BlockSpec((1, D), lambda e: (0, 0))
    in_specs = [full2d]
    if first:
        in_specs += [full2d, row, row]
    in_specs += [
        pl.BlockSpec((D, G1), lambda e: (0, 0)),
        pl.BlockSpec((1, G1), lambda e: (0, 0)),
        pl.BlockSpec((D, G2), lambda e: (0, 0)),
        pl.BlockSpec((1, G2), lambda e: (0, 0)),
        pl.BlockSpec((1, D, D), lambda e: (e, 0, 0)),
        pl.BlockSpec((1, 1, D), lambda e: (e, 0, 0)),
        pl.BlockSpec((1, D, D), lambda e: (e, 0, 0)),
        pl.BlockSpec((1, 1, D), lambda e: (e, 0, 0)),
        pl.BlockSpec((S, 1), lambda e: (0, 0)),
    ]
    scratch = []
    if first:
        scratch.append(pltpu.VMEM((S, D), jnp.float32))
    scratch.append(pltpu.VMEM((S, E), jnp.float32))
    return dict(
        grid=(E,),
        in_specs=in_specs,
        out_specs=full2d,
        out_shape=jax.ShapeDtypeStruct((S, D), jnp.float32),
        scratch_shapes=scratch,
    )


def _head_body(x_ref, hw_ref, hb_ref, hg_ref, hbe_ref, dw_ref, db_ref,
               lab_ref, logits_ref, loss_ref, h_s, m_s, s_s, ll_s):
    j = pl.program_id(0)

    @pl.when(j == 0)
    def _prologue():
        hh = jnp.dot(x_ref[...], hw_ref[...],
                     preferred_element_type=jnp.float32) + hb_ref[...]
        hh = jax.nn.gelu(hh)
        mu = jnp.mean(hh, axis=1, keepdims=True)
        var = jnp.mean((hh - mu) ** 2, axis=1, keepdims=True)
        h_s[...] = ((hh - mu) * lax.rsqrt(var + 1e-5) * hg_ref[...]
                    + hbe_ref[...])
        m_s[...] = jnp.full((S, 1), -jnp.inf, jnp.float32)
        s_s[...] = jnp.zeros((S, 1), jnp.float32)
        ll_s[...] = jnp.zeros((S, 1), jnp.float32)

    logits = jnp.dot(h_s[...], dw_ref[...],
                     preferred_element_type=jnp.float32) + db_ref[...]
    logits_ref[...] = logits
    colg = j * VT + lax.broadcasted_iota(jnp.int32, (S, VT), 1)
    lg = jnp.where(colg < V, logits, -jnp.inf)
    m_old = m_s[...]
    m_new = jnp.maximum(m_old, jnp.max(lg, axis=1, keepdims=True))
    s_s[...] = (s_s[...] * jnp.exp(m_old - m_new)
                + jnp.sum(jnp.exp(lg - m_new), axis=1, keepdims=True))
    m_s[...] = m_new
    ll_s[...] += jnp.sum(jnp.where(colg == lab_ref[...], logits, 0.0),
                         axis=1, keepdims=True)

    @pl.when(j == NV - 1)
    def _fin():
        nll = m_s[...] + jnp.log(s_s[...]) - ll_s[...]
        loss_ref[...] = jnp.sum(nll, keepdims=True) / S


def _head_pallas_args():
    row = pl.BlockSpec((1, D), lambda j: (0, 0))
    return dict(
        grid=(NV,),
        in_specs=[
            pl.BlockSpec((S, D), lambda j: (0, 0)),
            pl.BlockSpec((D, D), lambda j: (0, 0)),
            row, row, row,
            pl.BlockSpec((D, VT), lambda j: (0, j)),
            pl.BlockSpec((1, VT), lambda j: (0, j)),
            pl.BlockSpec((S, 1), lambda j: (0, 0)),
        ],
        out_specs=[
            pl.BlockSpec((S, VT), lambda j: (0, j)),
            pl.BlockSpec((1, 1), lambda j: (0, 0)),
        ],
        out_shape=[
            jax.ShapeDtypeStruct((S, V), jnp.float32),
            jax.ShapeDtypeStruct((1, 1), jnp.float32),
        ],
        scratch_shapes=[
            pltpu.VMEM((S, D), jnp.float32),
            pltpu.VMEM((S, 1), jnp.float32),
            pltpu.VMEM((S, 1), jnp.float32),
            pltpu.VMEM((S, 1), jnp.float32),
        ],
    )


def _forward(gather_fn, moe1, moe2, head, input_ids, attention_mask, labels,
             word_emb, pos_emb, type_emb, emb_ln_g, emb_ln_b, Wg1, bg1, Wg2,
             bg2, W1, b1, W2, b2, head_w, head_b, head_ln_g, head_ln_b,
             dec_w, dec_b):
    ids = input_ids.reshape(S)
    gathered = gather_fn(word_emb, ids)
    pos_ids = jnp.clip(jnp.arange(S) + 2, 0, pos_emb.shape[0] - 1)
    const = pos_emb[pos_ids] + type_emb[0][None, :]
    mask = attention_mask.reshape(S, 1).astype(jnp.float32)

    x = moe1(gathered, const, emb_ln_g.reshape(1, D), emb_ln_b.reshape(1, D),
             Wg1[0], bg1[0].reshape(1, G1), Wg2[0], bg2[0].reshape(1, G2),
             W1[0], b1[0].reshape(E, 1, D), W2[0], b2[0].reshape(E, 1, D),
             mask)
    x = moe2(x, Wg1[1], bg1[1].reshape(1, G1), Wg2[1], bg2[1].reshape(1, G2),
             W1[1], b1[1].reshape(E, 1, D), W2[1], b2[1].reshape(E, 1, D),
             mask)

    logits, loss11 = head(x, head_w, head_b.reshape(1, D),
                          head_ln_g.reshape(1, D),
                          head_ln_b.reshape(1, D), dec_w,
                          dec_b.reshape(1, V), labels.reshape(S, 1))
    return (loss11[0, 0], logits.reshape(1, S, V), x.reshape(1, S, D))


def kernel(input_ids, attention_mask, labels, word_emb, pos_emb, type_emb,
           emb_ln_g, emb_ln_b, Wg1, bg1, Wg2, bg2, W1, b1, W2, b2, head_w,
           head_b, head_ln_g, head_ln_b, dec_w, dec_b):
    gather_fn = _make_sc_gather()
    moe1 = pl.pallas_call(_make_moe_body(True), **_moe_pallas_args(True))
    moe2 = pl.pallas_call(_make_moe_body(False), **_moe_pallas_args(False))
    head = pl.pallas_call(_head_body, **_head_pallas_args())
    return _forward(gather_fn, moe1, moe2, head, input_ids, attention_mask,
                    labels, word_emb, pos_emb, type_emb, emb_ln_g, emb_ln_b,
                    Wg1, bg1, Wg2, bg2, W1, b1, W2, b2, head_w, head_b,
                    head_ln_g, head_ln_b, dec_w, dec_b)
